# baseline (speedup 1.0000x reference)
import functools

import jax
import jax.numpy as jnp
from jax import lax
from jax.experimental import pallas as pl
from jax.experimental.pallas import tpu as pltpu
from jax.experimental.pallas import tpu_sc as plsc

_N = 10000
_E = 320000
_NC = 2
_NS = 16
_CH = 125
_OB = 16
_RPT = 624
_CHUNKS = _E // _CH
_BN = 2000
_NBLK = _N // _BN

_DOT_KW = dict(preferred_element_type=jnp.float32,
               precision=lax.Precision.DEFAULT)
_DN = (((1,), (0,)), ((), ()))


def _sc_segment_sum(hs_rows, edge_split):
    cpc = _CHUNKS // 2 if edge_split else _CHUNKS
    cpt = cpc // _NS
    ob = 16 if edge_split else 32
    mesh = plsc.VectorSubcoreMesh(core_axis_name="c", subcore_axis_name="s")

    @functools.partial(
        pl.kernel,
        out_type=jax.ShapeDtypeStruct((2 * _N, 128), jnp.float32),
        mesh=mesh,
        scratch_types=[
            pltpu.VMEM((ob, _CH), jnp.int32),
            pltpu.VMEM((ob, _CH), jnp.int32),
            pltpu.VMEM((_CH, 128), jnp.float32),
            pltpu.VMEM((_CH, 128), jnp.float32),
            pltpu.VMEM_SHARED((_N, 128), jnp.float32),
            pltpu.SemaphoreType.DMA,
            pltpu.SemaphoreType.DMA,
            pltpu.SemaphoreType.DMA,
        ],
    )
    def seg(hs_hbm, src_hbm, dst_hbm, z_hbm, out_hbm,
            sidx, didx, rows0, rows1, acc, gsem0, gsem1, isem):
        c = lax.axis_index("c")
        s = lax.axis_index("s")
        r0 = s * _RPT
        pltpu.sync_copy(z_hbm.at[pl.ds(r0, _RPT)], acc.at[pl.ds(r0, _RPT)])

        @pl.when(s == _NS - 1)
        def _():
            pltpu.sync_copy(z_hbm.at[pl.ds(_NS * _RPT, _N - _NS * _RPT)],
                            acc.at[pl.ds(_NS * _RPT, _N - _NS * _RPT)])

        sbase = c * cpc + s * cpt
        dbase = (c * cpc + s * cpt) if edge_split else (s * cpt)
        plsc.subcore_barrier()

        rows = (rows0, rows1)
        sems = (gsem0, gsem1)

        @pl.loop(0, cpt // ob)
        def _(t):
            ip = pltpu.async_copy(src_hbm.at[pl.ds(sbase + t * ob, ob)],
                                  sidx, isem)
            pltpu.sync_copy(dst_hbm.at[pl.ds(dbase + t * ob, ob)], didx)
            ip.wait()
            pend = [pltpu.async_copy(hs_hbm.at[sidx.at[0]], rows[0], sems[0]),
                    None]
            for j in range(ob):
                if j + 1 < ob:
                    b = (j + 1) % 2
                    pend[b] = pltpu.async_copy(hs_hbm.at[sidx.at[j + 1]],
                                               rows[b], sems[b])
                pend[j % 2].wait()
                pltpu.sync_copy(rows[j % 2], acc.at[didx.at[j]], add=True)

        plsc.subcore_barrier()
        pltpu.sync_copy(acc.at[pl.ds(r0, _RPT)],
                        out_hbm.at[pl.ds(c * _N + r0, _RPT)])

        @pl.when(s == _NS - 1)
        def _():
            pltpu.sync_copy(
                acc.at[pl.ds(_NS * _RPT, _N - _NS * _RPT)],
                out_hbm.at[pl.ds(c * _N + _NS * _RPT, _N - _NS * _RPT)])

    def call(hs, src2, dst2, z):
        assert hs.shape == (hs_rows, 128)
        return seg(hs, src2, dst2, z)

    return call


def _tc_abs(x):
    def body(x_ref, o_ref):
        o_ref[...] = jnp.abs(x_ref[...])

    return pl.pallas_call(
        body,
        grid=(_NBLK,),
        in_specs=[pl.BlockSpec((_BN, 128), lambda i: (i, 0))],
        out_specs=pl.BlockSpec((_BN, 128), lambda i: (i, 0)),
        out_shape=jax.ShapeDtypeStruct((_N, 128), jnp.float32),
    )(x)


def _tc_hw(hs, w, stacked):
    if stacked:
        def body(hlo, hhi, w_ref, o_ref):
            acc = lax.dot_general(hlo[...], w_ref[pl.ds(0, 128), :],
                                  _DN, **_DOT_KW)
            acc += lax.dot_general(hhi[...], w_ref[pl.ds(128, 128), :],
                                   _DN, **_DOT_KW)
            o_ref[...] = acc

        in_specs = [
            pl.BlockSpec((_BN, 128), lambda j, i: (i, 0)),
            pl.BlockSpec((_BN, 128), lambda j, i: (_NBLK + i, 0)),
            pl.BlockSpec((256, 128), lambda j, i: (0, j)),
        ]
        args = (hs, hs, w)
    else:
        def body(h_ref, w_ref, o_ref):
            o_ref[...] = lax.dot_general(h_ref[...], w_ref[...],
                                         _DN, **_DOT_KW)

        in_specs = [
            pl.BlockSpec((_BN, 128), lambda j, i: (i, 0)),
            pl.BlockSpec((128, 128), lambda j, i: (0, j)),
        ]
        args = (hs, w)

    return pl.pallas_call(
        body,
        grid=(2, _NBLK),
        in_specs=in_specs,
        out_specs=pl.BlockSpec((_BN, 128), lambda j, i: (j * _NBLK + i, 0)),
        out_shape=jax.ShapeDtypeStruct((2 * _N, 128), jnp.float32),
    )(*args)


def _tc_comb(aggs, hw, wd, edge_split):
    if edge_split:
        def body(alo, ahi, hw_ref, wd_ref, o_ref):
            acc = lax.dot_general(alo[...] + ahi[...], wd_ref[...],
                                  _DN, **_DOT_KW)
            o_ref[...] = jnp.maximum(acc + hw_ref[...], 0.0)

        wd_spec = pl.BlockSpec((128, 128), lambda j, i: (0, j))
    else:
        def body(alo, ahi, hw_ref, wd_ref, o_ref):
            acc = lax.dot_general(alo[...], wd_ref[pl.ds(0, 128), :],
                                  _DN, **_DOT_KW)
            acc += lax.dot_general(ahi[...], wd_ref[pl.ds(128, 128), :],
                                   _DN, **_DOT_KW)
            o_ref[...] = jnp.maximum(acc + hw_ref[...], 0.0)

        wd_spec = pl.BlockSpec((256, 128), lambda j, i: (0, j))

    return pl.pallas_call(
        body,
        grid=(2, _NBLK),
        in_specs=[
            pl.BlockSpec((_BN, 128), lambda j, i: (i, 0)),
            pl.BlockSpec((_BN, 128), lambda j, i: (_NBLK + i, 0)),
            pl.BlockSpec((_BN, 128), lambda j, i: (j * _NBLK + i, 0)),
            wd_spec,
        ],
        out_specs=pl.BlockSpec((_BN, 128), lambda j, i: (j * _NBLK + i, 0)),
        out_shape=jax.ShapeDtypeStruct((2 * _N, 128), jnp.float32),
    )(aggs, aggs, hw, wd)


def _tc_pool_head(hs3, batch2d, w1, b1, w2, b2):
    def body(h_ref, b_ref, w1_ref, b1_ref, w2_ref, b2_ref, o_ref, pool_scr):
        j = pl.program_id(0)
        i = pl.program_id(1)

        @pl.when((j == 0) & (i == 0))
        def _():
            pool_scr[...] = jnp.zeros_like(pool_scr)

        oh = (b_ref[...] == lax.broadcasted_iota(jnp.int32, (_BN, 64), 1))
        pool_scr[j] += lax.dot_general(oh.astype(jnp.float32), h_ref[...],
                                       (((0,), (0,)), ((), ())), **_DOT_KW)

        @pl.when((j == 1) & (i == _NBLK - 1))
        def _():
            p = jnp.concatenate([pool_scr[0], pool_scr[1]], axis=1)
            t = lax.dot_general(p, w1_ref[...], _DN, **_DOT_KW) + b1_ref[...]
            t = jnp.maximum(t, 0.0)
            o_ref[...] = lax.dot_general(t, w2_ref[...], _DN,
                                         **_DOT_KW) + b2_ref[...]

    return pl.pallas_call(
        body,
        grid=(2, _NBLK),
        in_specs=[
            pl.BlockSpec((_BN, 128), lambda j, i: (j * _NBLK + i, 0)),
            pl.BlockSpec((_BN, 1), lambda j, i: (i, 0)),
            pl.BlockSpec((256, 256), lambda j, i: (0, 0)),
            pl.BlockSpec((1, 256), lambda j, i: (0, 0)),
            pl.BlockSpec((256, 10), lambda j, i: (0, 0)),
            pl.BlockSpec((1, 10), lambda j, i: (0, 0)),
        ],
        out_specs=pl.BlockSpec((64, 10), lambda j, i: (0, 0)),
        out_shape=jax.ShapeDtypeStruct((64, 10), jnp.float32),
        scratch_shapes=[pltpu.VMEM((2, 64, 128), jnp.float32)],
    )(hs3, batch2d, w1, b1, w2, b2)


def kernel(x, lower_index, batch, W_down_0, W_0, W_down_1, W_1,
           W_down_2, W_2, lin1_w, lin1_b, lin2_w, lin2_b):
    src = lower_index[0]
    dst = lower_index[1]
    src2 = jnp.concatenate([src, src + _N]).reshape(2 * _CHUNKS, _CH)
    dst2 = dst.reshape(_CHUNKS, _CH)
    z128 = jnp.zeros((_N, 128), jnp.float32)
    batch2d = batch.reshape(_N, 1)
    b1 = lin1_b.reshape(1, 256)
    b2 = lin2_b.reshape(1, 10)

    seg_e = _sc_segment_sum(_N, edge_split=True)
    seg_f = _sc_segment_sum(2 * _N, edge_split=False)

    h0 = _tc_abs(x)
    a0 = seg_e(h0, src2, dst2, z128)
    hw0 = _tc_hw(h0, W_0, stacked=False)
    hs1 = _tc_comb(a0, hw0, W_down_0, edge_split=True)
    a1 = seg_f(hs1, src2, dst2, z128)
    hw1 = _tc_hw(hs1, W_1, stacked=True)
    hs2 = _tc_comb(a1, hw1, W_down_1, edge_split=False)
    a2 = seg_f(hs2, src2, dst2, z128)
    hw2 = _tc_hw(hs2, W_2, stacked=True)
    hs3 = _tc_comb(a2, hw2, W_down_2, edge_split=False)
    return _tc_pool_head(hs3, batch2d, lin1_w, b1, lin2_w, b2)


# speedup vs baseline: 1.0207x; 1.0207x over previous
import functools

import jax
import jax.numpy as jnp
from jax import lax
from jax.experimental import pallas as pl
from jax.experimental.pallas import tpu as pltpu
from jax.experimental.pallas import tpu_sc as plsc

_N = 10000
_E = 320000
_NC = 2
_NS = 16
_CH = 125
_OB = 16
_RPT = 624
_CHUNKS = _E // _CH
_BN = 2000
_NBLK = _N // _BN

_DOT_KW = dict(preferred_element_type=jnp.float32,
               precision=lax.Precision.DEFAULT)
_DN = (((1,), (0,)), ((), ()))


def _sc_segment_sum(hs_rows, edge_split):
    cpc = _CHUNKS // 2 if edge_split else _CHUNKS
    cpt = cpc // _NS
    ob = 16 if edge_split else 32
    mesh = plsc.VectorSubcoreMesh(core_axis_name="c", subcore_axis_name="s")

    @functools.partial(
        pl.kernel,
        out_type=jax.ShapeDtypeStruct((2 * _N, 128), jnp.float32),
        mesh=mesh,
        scratch_types=[
            pltpu.VMEM((ob, _CH), jnp.int32),
            pltpu.VMEM((ob, _CH), jnp.int32),
            pltpu.VMEM((_CH, 128), jnp.float32),
            pltpu.VMEM((_CH, 128), jnp.float32),
            pltpu.VMEM_SHARED((_N, 128), jnp.float32),
            pltpu.SemaphoreType.DMA,
            pltpu.SemaphoreType.DMA,
            pltpu.SemaphoreType.DMA,
        ],
    )
    def seg(hs_hbm, src_hbm, dst_hbm, z_hbm, out_hbm,
            sidx, didx, rows0, rows1, acc, gsem0, gsem1, isem):
        c = lax.axis_index("c")
        s = lax.axis_index("s")
        r0 = s * _RPT
        pltpu.sync_copy(z_hbm.at[pl.ds(r0, _RPT)], acc.at[pl.ds(r0, _RPT)])

        @pl.when(s == _NS - 1)
        def _():
            pltpu.sync_copy(z_hbm.at[pl.ds(_NS * _RPT, _N - _NS * _RPT)],
                            acc.at[pl.ds(_NS * _RPT, _N - _NS * _RPT)])

        sbase = c * cpc + s * cpt
        dbase = (c * cpc + s * cpt) if edge_split else (s * cpt)
        plsc.subcore_barrier()

        rows = (rows0, rows1)
        sems = (gsem0, gsem1)

        @pl.loop(0, cpt // ob)
        def _(t):
            ip = pltpu.async_copy(src_hbm.at[pl.ds(sbase + t * ob, ob)],
                                  sidx, isem)
            pltpu.sync_copy(dst_hbm.at[pl.ds(dbase + t * ob, ob)], didx)
            ip.wait()
            pend = [pltpu.async_copy(hs_hbm.at[sidx.at[0]], rows[0], sems[0]),
                    None]
            for j in range(ob):
                if j + 1 < ob:
                    b = (j + 1) % 2
                    pend[b] = pltpu.async_copy(hs_hbm.at[sidx.at[j + 1]],
                                               rows[b], sems[b])
                pend[j % 2].wait()
                pltpu.sync_copy(rows[j % 2], acc.at[didx.at[j]], add=True)

        plsc.subcore_barrier()
        pltpu.sync_copy(acc.at[pl.ds(r0, _RPT)],
                        out_hbm.at[pl.ds(c * _N + r0, _RPT)])

        @pl.when(s == _NS - 1)
        def _():
            pltpu.sync_copy(
                acc.at[pl.ds(_NS * _RPT, _N - _NS * _RPT)],
                out_hbm.at[pl.ds(c * _N + _NS * _RPT, _N - _NS * _RPT)])

    def call(hs, src2, dst2, z):
        assert hs.shape == (hs_rows, 128)
        return seg(hs, src2, dst2, z)

    return call


def _tc_abs(x):
    def body(x_ref, o_ref):
        o_ref[...] = jnp.abs(x_ref[...])

    return pl.pallas_call(
        body,
        grid=(_NBLK,),
        in_specs=[pl.BlockSpec((_BN, 128), lambda i: (i, 0))],
        out_specs=pl.BlockSpec((_BN, 128), lambda i: (i, 0)),
        out_shape=jax.ShapeDtypeStruct((_N, 128), jnp.float32),
    )(x)


def _tc_layer0(aggp, h0, wd, w):
    def body(alo, ahi, h_ref, wd_ref, w_ref, o_ref):
        acc = lax.dot_general(alo[...] + ahi[...], wd_ref[...], _DN, **_DOT_KW)
        acc += lax.dot_general(h_ref[...], w_ref[...], _DN, **_DOT_KW)
        o_ref[...] = jnp.maximum(acc, 0.0)

    return pl.pallas_call(
        body,
        grid=(2, _NBLK),
        in_specs=[
            pl.BlockSpec((_BN, 128), lambda j, i: (i, 0)),
            pl.BlockSpec((_BN, 128), lambda j, i: (_NBLK + i, 0)),
            pl.BlockSpec((_BN, 128), lambda j, i: (i, 0)),
            pl.BlockSpec((128, 128), lambda j, i: (0, j)),
            pl.BlockSpec((128, 128), lambda j, i: (0, j)),
        ],
        out_specs=pl.BlockSpec((_BN, 128), lambda j, i: (j * _NBLK + i, 0)),
        out_shape=jax.ShapeDtypeStruct((2 * _N, 128), jnp.float32),
    )(aggp, aggp, h0, wd, w)


def _tc_layer(aggs, hs, wd, w):
    def body(alo, ahi, hlo, hhi, wd_ref, w_ref, o_ref):
        acc = lax.dot_general(alo[...], wd_ref[pl.ds(0, 128), :],
                              _DN, **_DOT_KW)
        acc += lax.dot_general(ahi[...], wd_ref[pl.ds(128, 128), :],
                               _DN, **_DOT_KW)
        acc += lax.dot_general(hlo[...], w_ref[pl.ds(0, 128), :],
                               _DN, **_DOT_KW)
        acc += lax.dot_general(hhi[...], w_ref[pl.ds(128, 128), :],
                               _DN, **_DOT_KW)
        o_ref[...] = jnp.maximum(acc, 0.0)

    return pl.pallas_call(
        body,
        grid=(2, _NBLK),
        in_specs=[
            pl.BlockSpec((_BN, 128), lambda j, i: (i, 0)),
            pl.BlockSpec((_BN, 128), lambda j, i: (_NBLK + i, 0)),
            pl.BlockSpec((_BN, 128), lambda j, i: (i, 0)),
            pl.BlockSpec((_BN, 128), lambda j, i: (_NBLK + i, 0)),
            pl.BlockSpec((256, 128), lambda j, i: (0, j)),
            pl.BlockSpec((256, 128), lambda j, i: (0, j)),
        ],
        out_specs=pl.BlockSpec((_BN, 128), lambda j, i: (j * _NBLK + i, 0)),
        out_shape=jax.ShapeDtypeStruct((2 * _N, 128), jnp.float32),
    )(aggs, aggs, hs, hs, wd, w)


def _tc_layer_pool_head(aggs, hs, wd, w, batch2d, w1, b1, w2, b2):
    def body(alo, ahi, hlo, hhi, wd_ref, w_ref, b_ref,
             w1_ref, b1_ref, w2_ref, b2_ref, o_ref, pool_scr):
        j = pl.program_id(0)
        i = pl.program_id(1)

        acc = lax.dot_general(alo[...], wd_ref[pl.ds(0, 128), :],
                              _DN, **_DOT_KW)
        acc += lax.dot_general(ahi[...], wd_ref[pl.ds(128, 128), :],
                               _DN, **_DOT_KW)
        acc += lax.dot_general(hlo[...], w_ref[pl.ds(0, 128), :],
                               _DN, **_DOT_KW)
        acc += lax.dot_general(hhi[...], w_ref[pl.ds(128, 128), :],
                               _DN, **_DOT_KW)
        acc = jnp.maximum(acc, 0.0)

        @pl.when((j == 0) & (i == 0))
        def _():
            pool_scr[...] = jnp.zeros_like(pool_scr)

        oh = (b_ref[...] == lax.broadcasted_iota(jnp.int32, (_BN, 64), 1))
        pool_scr[j] += lax.dot_general(oh.astype(jnp.float32), acc,
                                       (((0,), (0,)), ((), ())), **_DOT_KW)

        @pl.when((j == 1) & (i == _NBLK - 1))
        def _():
            p = jnp.concatenate([pool_scr[0], pool_scr[1]], axis=1)
            t = lax.dot_general(p, w1_ref[...], _DN, **_DOT_KW) + b1_ref[...]
            t = jnp.maximum(t, 0.0)
            o_ref[...] = lax.dot_general(t, w2_ref[...], _DN,
                                         **_DOT_KW) + b2_ref[...]

    return pl.pallas_call(
        body,
        grid=(2, _NBLK),
        in_specs=[
            pl.BlockSpec((_BN, 128), lambda j, i: (i, 0)),
            pl.BlockSpec((_BN, 128), lambda j, i: (_NBLK + i, 0)),
            pl.BlockSpec((_BN, 128), lambda j, i: (i, 0)),
            pl.BlockSpec((_BN, 128), lambda j, i: (_NBLK + i, 0)),
            pl.BlockSpec((256, 128), lambda j, i: (0, j)),
            pl.BlockSpec((256, 128), lambda j, i: (0, j)),
            pl.BlockSpec((_BN, 1), lambda j, i: (i, 0)),
            pl.BlockSpec((256, 256), lambda j, i: (0, 0)),
            pl.BlockSpec((1, 256), lambda j, i: (0, 0)),
            pl.BlockSpec((256, 10), lambda j, i: (0, 0)),
            pl.BlockSpec((1, 10), lambda j, i: (0, 0)),
        ],
        out_specs=pl.BlockSpec((64, 10), lambda j, i: (0, 0)),
        out_shape=jax.ShapeDtypeStruct((64, 10), jnp.float32),
        scratch_shapes=[pltpu.VMEM((2, 64, 128), jnp.float32)],
    )(aggs, aggs, hs, hs, wd, w, batch2d, w1, b1, w2, b2)


def kernel(x, lower_index, batch, W_down_0, W_0, W_down_1, W_1,
           W_down_2, W_2, lin1_w, lin1_b, lin2_w, lin2_b):
    src = lower_index[0]
    dst = lower_index[1]
    src2 = jnp.concatenate([src, src + _N]).reshape(2 * _CHUNKS, _CH)
    dst2 = dst.reshape(_CHUNKS, _CH)
    z128 = jnp.zeros((_N, 128), jnp.float32)
    batch2d = batch.reshape(_N, 1)
    b1 = lin1_b.reshape(1, 256)
    b2 = lin2_b.reshape(1, 10)

    seg_e = _sc_segment_sum(_N, edge_split=True)
    seg_f = _sc_segment_sum(2 * _N, edge_split=False)

    h0 = _tc_abs(x)
    a0 = seg_e(h0, src2, dst2, z128)
    hs1 = _tc_layer0(a0, h0, W_down_0, W_0)
    a1 = seg_f(hs1, src2, dst2, z128)
    hs2 = _tc_layer(a1, hs1, W_down_1, W_1)
    a2 = seg_f(hs2, src2, dst2, z128)
    return _tc_layer_pool_head(a2, hs2, W_down_2, W_2, batch2d,
                               lin1_w, b1, lin2_w, b2)


# speedup vs baseline: 1.0749x; 1.0531x over previous
import functools

import jax
import jax.numpy as jnp
from jax import lax
from jax.experimental import pallas as pl
from jax.experimental.pallas import tpu as pltpu
from jax.experimental.pallas import tpu_sc as plsc

_N = 10000
_E = 320000
_NC = 2
_NS = 16
_CH = 125
_OB = 16
_RPT = 624
_CHUNKS = _E // _CH
_BN = 2000
_NBLK = _N // _BN

_DOT_KW = dict(preferred_element_type=jnp.float32,
               precision=lax.Precision.DEFAULT)
_DN = (((1,), (0,)), ((), ()))


def _sc_segment_sum(hs_rows, edge_split):
    cpc = _CHUNKS // 2 if edge_split else _CHUNKS
    cpt = cpc // _NS
    ob = 8
    nb = cpt // ob
    mesh = plsc.VectorSubcoreMesh(core_axis_name="c", subcore_axis_name="s")

    @functools.partial(
        pl.kernel,
        out_type=jax.ShapeDtypeStruct((2 * _N, 128), jnp.float32),
        mesh=mesh,
        scratch_types=[
            pltpu.VMEM((2, ob, _CH), jnp.int32),
            pltpu.VMEM((2, ob, _CH), jnp.int32),
            pltpu.VMEM((_CH, 128), jnp.float32),
            pltpu.VMEM((_CH, 128), jnp.float32),
            pltpu.VMEM_SHARED((_N, 128), jnp.float32),
            pltpu.SemaphoreType.DMA,
            pltpu.SemaphoreType.DMA,
            pltpu.SemaphoreType.DMA,
            pltpu.SemaphoreType.DMA,
        ],
    )
    def seg(hs_hbm, src_hbm, dst_hbm, z_hbm, out_hbm,
            sidx2, didx2, rows0, rows1, acc, gsem0, gsem1, isem0, isem1):
        c = lax.axis_index("c")
        s = lax.axis_index("s")
        r0 = s * _RPT
        sbase = c * cpc + s * cpt
        dbase = (c * cpc + s * cpt) if edge_split else (s * cpt)
        rows = (rows0, rows1)
        gsems = (gsem0, gsem1)
        isems = (isem0, isem1)

        def stage(blk, p):
            pltpu.async_copy(src_hbm.at[pl.ds(sbase + blk * ob, ob)],
                             sidx2.at[p], isems[p])
            pltpu.async_copy(dst_hbm.at[pl.ds(dbase + blk * ob, ob)],
                             didx2.at[p], isems[p])

        def stage_wait(blk, p):
            pltpu.make_async_copy(src_hbm.at[pl.ds(sbase + blk * ob, ob)],
                                  sidx2.at[p], isems[p]).wait()
            pltpu.make_async_copy(dst_hbm.at[pl.ds(dbase + blk * ob, ob)],
                                  didx2.at[p], isems[p]).wait()

        def gather(p, j, jb):
            pltpu.async_copy(hs_hbm.at[sidx2.at[p].at[j]], rows[jb],
                             gsems[jb])

        def gather_wait(p, j, jb):
            pltpu.make_async_copy(hs_hbm.at[sidx2.at[p].at[j]], rows[jb],
                                  gsems[jb]).wait()

        stage(0, 0)
        stage_wait(0, 0)
        gather(0, 0, 0)
        gather(0, 1, 1)
        pltpu.sync_copy(z_hbm.at[pl.ds(r0, _RPT)], acc.at[pl.ds(r0, _RPT)])

        @pl.when(s == _NS - 1)
        def _():
            pltpu.sync_copy(z_hbm.at[pl.ds(_NS * _RPT, _N - _NS * _RPT)],
                            acc.at[pl.ds(_NS * _RPT, _N - _NS * _RPT)])

        plsc.subcore_barrier()

        def one_block(blk, p):
            dx = didx2.at[p]
            for j in range(ob):
                jb = j % 2
                if j == 0:
                    @pl.when(blk + 1 < nb)
                    def _():
                        stage(blk + 1, 1 - p)
                gather_wait(p, j, jb)
                pltpu.sync_copy(rows[jb], acc.at[dx.at[j]], add=True)
                if j + 2 < ob:
                    gather(p, j + 2, jb)
                else:
                    @pl.when(blk + 1 < nb)
                    def _():
                        if j + 2 == ob:
                            stage_wait(blk + 1, 1 - p)
                        gather(1 - p, j + 2 - ob, jb)

        @pl.loop(0, nb // 2)
        def _(m):
            one_block(2 * m, 0)
            one_block(2 * m + 1, 1)

        plsc.subcore_barrier()
        pltpu.sync_copy(acc.at[pl.ds(r0, _RPT)],
                        out_hbm.at[pl.ds(c * _N + r0, _RPT)])

        @pl.when(s == _NS - 1)
        def _():
            pltpu.sync_copy(
                acc.at[pl.ds(_NS * _RPT, _N - _NS * _RPT)],
                out_hbm.at[pl.ds(c * _N + _NS * _RPT, _N - _NS * _RPT)])

    def call(hs, src2, dst2, z):
        assert hs.shape == (hs_rows, 128)
        return seg(hs, src2, dst2, z)

    return call


def _tc_abs(x):
    def body(x_ref, o_ref):
        o_ref[...] = jnp.abs(x_ref[...])

    return pl.pallas_call(
        body,
        grid=(_NBLK,),
        in_specs=[pl.BlockSpec((_BN, 128), lambda i: (i, 0))],
        out_specs=pl.BlockSpec((_BN, 128), lambda i: (i, 0)),
        out_shape=jax.ShapeDtypeStruct((_N, 128), jnp.float32),
    )(x)


def _tc_layer0(aggp, h0, wd, w):
    def body(alo, ahi, h_ref, wd_ref, w_ref, o_ref):
        acc = lax.dot_general(alo[...] + ahi[...], wd_ref[...], _DN, **_DOT_KW)
        acc += lax.dot_general(h_ref[...], w_ref[...], _DN, **_DOT_KW)
        o_ref[...] = jnp.maximum(acc, 0.0)

    return pl.pallas_call(
        body,
        grid=(2, _NBLK),
        in_specs=[
            pl.BlockSpec((_BN, 128), lambda j, i: (i, 0)),
            pl.BlockSpec((_BN, 128), lambda j, i: (_NBLK + i, 0)),
            pl.BlockSpec((_BN, 128), lambda j, i: (i, 0)),
            pl.BlockSpec((128, 128), lambda j, i: (0, j)),
            pl.BlockSpec((128, 128), lambda j, i: (0, j)),
        ],
        out_specs=pl.BlockSpec((_BN, 128), lambda j, i: (j * _NBLK + i, 0)),
        out_shape=jax.ShapeDtypeStruct((2 * _N, 128), jnp.float32),
    )(aggp, aggp, h0, wd, w)


def _tc_layer(aggs, hs, wd, w):
    def body(alo, ahi, hlo, hhi, wd_ref, w_ref, o_ref):
        acc = lax.dot_general(alo[...], wd_ref[pl.ds(0, 128), :],
                              _DN, **_DOT_KW)
        acc += lax.dot_general(ahi[...], wd_ref[pl.ds(128, 128), :],
                               _DN, **_DOT_KW)
        acc += lax.dot_general(hlo[...], w_ref[pl.ds(0, 128), :],
                               _DN, **_DOT_KW)
        acc += lax.dot_general(hhi[...], w_ref[pl.ds(128, 128), :],
                               _DN, **_DOT_KW)
        o_ref[...] = jnp.maximum(acc, 0.0)

    return pl.pallas_call(
        body,
        grid=(2, _NBLK),
        in_specs=[
            pl.BlockSpec((_BN, 128), lambda j, i: (i, 0)),
            pl.BlockSpec((_BN, 128), lambda j, i: (_NBLK + i, 0)),
            pl.BlockSpec((_BN, 128), lambda j, i: (i, 0)),
            pl.BlockSpec((_BN, 128), lambda j, i: (_NBLK + i, 0)),
            pl.BlockSpec((256, 128), lambda j, i: (0, j)),
            pl.BlockSpec((256, 128), lambda j, i: (0, j)),
        ],
        out_specs=pl.BlockSpec((_BN, 128), lambda j, i: (j * _NBLK + i, 0)),
        out_shape=jax.ShapeDtypeStruct((2 * _N, 128), jnp.float32),
    )(aggs, aggs, hs, hs, wd, w)


def _tc_layer_pool_head(aggs, hs, wd, w, batch2d, w1, b1, w2, b2):
    def body(alo, ahi, hlo, hhi, wd_ref, w_ref, b_ref,
             w1_ref, b1_ref, w2_ref, b2_ref, o_ref, pool_scr):
        j = pl.program_id(0)
        i = pl.program_id(1)

        acc = lax.dot_general(alo[...], wd_ref[pl.ds(0, 128), :],
                              _DN, **_DOT_KW)
        acc += lax.dot_general(ahi[...], wd_ref[pl.ds(128, 128), :],
                               _DN, **_DOT_KW)
        acc += lax.dot_general(hlo[...], w_ref[pl.ds(0, 128), :],
                               _DN, **_DOT_KW)
        acc += lax.dot_general(hhi[...], w_ref[pl.ds(128, 128), :],
                               _DN, **_DOT_KW)
        acc = jnp.maximum(acc, 0.0)

        @pl.when((j == 0) & (i == 0))
        def _():
            pool_scr[...] = jnp.zeros_like(pool_scr)

        oh = (b_ref[...] == lax.broadcasted_iota(jnp.int32, (_BN, 64), 1))
        pool_scr[j] += lax.dot_general(oh.astype(jnp.float32), acc,
                                       (((0,), (0,)), ((), ())), **_DOT_KW)

        @pl.when((j == 1) & (i == _NBLK - 1))
        def _():
            p = jnp.concatenate([pool_scr[0], pool_scr[1]], axis=1)
            t = lax.dot_general(p, w1_ref[...], _DN, **_DOT_KW) + b1_ref[...]
            t = jnp.maximum(t, 0.0)
            o_ref[...] = lax.dot_general(t, w2_ref[...], _DN,
                                         **_DOT_KW) + b2_ref[...]

    return pl.pallas_call(
        body,
        grid=(2, _NBLK),
        in_specs=[
            pl.BlockSpec((_BN, 128), lambda j, i: (i, 0)),
            pl.BlockSpec((_BN, 128), lambda j, i: (_NBLK + i, 0)),
            pl.BlockSpec((_BN, 128), lambda j, i: (i, 0)),
            pl.BlockSpec((_BN, 128), lambda j, i: (_NBLK + i, 0)),
            pl.BlockSpec((256, 128), lambda j, i: (0, j)),
            pl.BlockSpec((256, 128), lambda j, i: (0, j)),
            pl.BlockSpec((_BN, 1), lambda j, i: (i, 0)),
            pl.BlockSpec((256, 256), lambda j, i: (0, 0)),
            pl.BlockSpec((1, 256), lambda j, i: (0, 0)),
            pl.BlockSpec((256, 10), lambda j, i: (0, 0)),
            pl.BlockSpec((1, 10), lambda j, i: (0, 0)),
        ],
        out_specs=pl.BlockSpec((64, 10), lambda j, i: (0, 0)),
        out_shape=jax.ShapeDtypeStruct((64, 10), jnp.float32),
        scratch_shapes=[pltpu.VMEM((2, 64, 128), jnp.float32)],
    )(aggs, aggs, hs, hs, wd, w, batch2d, w1, b1, w2, b2)


def kernel(x, lower_index, batch, W_down_0, W_0, W_down_1, W_1,
           W_down_2, W_2, lin1_w, lin1_b, lin2_w, lin2_b):
    src = lower_index[0]
    dst = lower_index[1]
    src2 = jnp.concatenate([src, src + _N]).reshape(2 * _CHUNKS, _CH)
    dst2 = dst.reshape(_CHUNKS, _CH)
    z128 = jnp.zeros((_N, 128), jnp.float32)
    batch2d = batch.reshape(_N, 1)
    b1 = lin1_b.reshape(1, 256)
    b2 = lin2_b.reshape(1, 10)

    seg_e = _sc_segment_sum(_N, edge_split=True)
    seg_f = _sc_segment_sum(2 * _N, edge_split=False)

    h0 = _tc_abs(x)
    a0 = seg_e(h0, src2, dst2, z128)
    hs1 = _tc_layer0(a0, h0, W_down_0, W_0)
    a1 = seg_f(hs1, src2, dst2, z128)
    hs2 = _tc_layer(a1, hs1, W_down_1, W_1)
    a2 = seg_f(hs2, src2, dst2, z128)
    return _tc_layer_pool_head(a2, hs2, W_down_2, W_2, batch2d,
                               lin1_w, b1, lin2_w, b2)


# speedup vs baseline: 1.1002x; 1.0236x over previous
import functools

import jax
import jax.numpy as jnp
from jax import lax
from jax.experimental import pallas as pl
from jax.experimental.pallas import tpu as pltpu
from jax.experimental.pallas import tpu_sc as plsc

_N = 10000
_E = 320000
_NC = 2
_NS = 16
_CH = 125
_OB = 16
_RPT = 624
_CHUNKS = _E // _CH
_BN = 2000
_NBLK = _N // _BN

_DOT_KW = dict(preferred_element_type=jnp.float32,
               precision=lax.Precision.DEFAULT)
_DN = (((1,), (0,)), ((), ()))


def _sc_segment_sum(hs_rows, edge_split):
    cpc = _CHUNKS // 2 if edge_split else _CHUNKS
    cpt = cpc // _NS
    ob = 8
    nb = cpt // ob
    mesh = plsc.VectorSubcoreMesh(core_axis_name="c", subcore_axis_name="s")

    @functools.partial(
        pl.kernel,
        out_type=jax.ShapeDtypeStruct((2 * _N, 128), jnp.float32),
        mesh=mesh,
        scratch_types=[
            pltpu.VMEM((2, ob, _CH), jnp.int32),
            pltpu.VMEM((2, ob, _CH), jnp.int32),
            pltpu.VMEM((_CH, 128), jnp.float32),
            pltpu.VMEM((_CH, 128), jnp.float32),
            pltpu.VMEM_SHARED((_N, 128), jnp.float32),
            pltpu.SemaphoreType.DMA,
            pltpu.SemaphoreType.DMA,
            pltpu.SemaphoreType.DMA,
            pltpu.SemaphoreType.DMA,
        ],
    )
    def seg(hs_hbm, src_hbm, dst_hbm, z_hbm, out_hbm,
            sidx2, didx2, rows0, rows1, acc, gsem0, gsem1, isem0, isem1):
        c = lax.axis_index("c")
        s = lax.axis_index("s")
        r0 = s * _RPT
        sbase = c * cpc + s * cpt
        dbase = (c * cpc + s * cpt) if edge_split else (s * cpt)
        rows = (rows0, rows1)
        gsems = (gsem0, gsem1)
        isems = (isem0, isem1)

        def stage(blk, p):
            pltpu.async_copy(src_hbm.at[pl.ds(sbase + blk * ob, ob)],
                             sidx2.at[p], isems[p])
            pltpu.async_copy(dst_hbm.at[pl.ds(dbase + blk * ob, ob)],
                             didx2.at[p], isems[p])

        def stage_wait(blk, p):
            pltpu.make_async_copy(src_hbm.at[pl.ds(sbase + blk * ob, ob)],
                                  sidx2.at[p], isems[p]).wait()
            pltpu.make_async_copy(dst_hbm.at[pl.ds(dbase + blk * ob, ob)],
                                  didx2.at[p], isems[p]).wait()

        def gather(p, j, jb):
            pltpu.async_copy(hs_hbm.at[sidx2.at[p].at[j]], rows[jb],
                             gsems[jb])

        def gather_wait(p, j, jb):
            pltpu.make_async_copy(hs_hbm.at[sidx2.at[p].at[j]], rows[jb],
                                  gsems[jb]).wait()

        stage(0, 0)
        stage_wait(0, 0)
        gather(0, 0, 0)
        gather(0, 1, 1)
        pltpu.sync_copy(z_hbm.at[pl.ds(r0, _RPT)], acc.at[pl.ds(r0, _RPT)])

        @pl.when(s == _NS - 1)
        def _():
            pltpu.sync_copy(z_hbm.at[pl.ds(_NS * _RPT, _N - _NS * _RPT)],
                            acc.at[pl.ds(_NS * _RPT, _N - _NS * _RPT)])

        plsc.subcore_barrier()

        def one_block(blk, p):
            dx = didx2.at[p]
            for j in range(ob):
                jb = j % 2
                if j == 0:
                    @pl.when(blk + 1 < nb)
                    def _():
                        stage(blk + 1, 1 - p)
                gather_wait(p, j, jb)
                pltpu.sync_copy(rows[jb], acc.at[dx.at[j]], add=True)
                if j + 2 < ob:
                    gather(p, j + 2, jb)
                else:
                    @pl.when(blk + 1 < nb)
                    def _():
                        if j + 2 == ob:
                            stage_wait(blk + 1, 1 - p)
                        gather(1 - p, j + 2 - ob, jb)

        @pl.loop(0, nb // 2)
        def _(m):
            one_block(2 * m, 0)
            one_block(2 * m + 1, 1)

        plsc.subcore_barrier()
        pltpu.sync_copy(acc.at[pl.ds(r0, _RPT)],
                        out_hbm.at[pl.ds(c * _N + r0, _RPT)])

        @pl.when(s == _NS - 1)
        def _():
            pltpu.sync_copy(
                acc.at[pl.ds(_NS * _RPT, _N - _NS * _RPT)],
                out_hbm.at[pl.ds(c * _N + _NS * _RPT, _N - _NS * _RPT)])

    def call(hs, src2, dst2, z):
        assert hs.shape == (hs_rows, 128)
        return seg(hs, src2, dst2, z)

    return call


def _tc_abs(x):
    def body(x_ref, o_ref):
        o_ref[...] = jnp.abs(x_ref[...])

    return pl.pallas_call(
        body,
        grid=(_NBLK,),
        in_specs=[pl.BlockSpec((_BN, 128), lambda i: (i, 0))],
        out_specs=pl.BlockSpec((_BN, 128), lambda i: (i, 0)),
        out_shape=jax.ShapeDtypeStruct((_N, 128), jnp.float32),
    )(x)


def _halves_matmul(a3, wref):
    acc = lax.dot_general(a3[0], wref[pl.ds(0, 128), :], _DN, **_DOT_KW)
    return acc + lax.dot_general(a3[1], wref[pl.ds(128, 128), :],
                                 _DN, **_DOT_KW)


def _tc_layer0(aggp, h0, wd, w):
    def body(a_ref, h_ref, wd_ref, w_ref, o_ref):
        agg = a_ref[0] + a_ref[1]
        acc = lax.dot_general(agg, wd_ref[...], _DN, **_DOT_KW)
        acc += lax.dot_general(h_ref[...], w_ref[...], _DN, **_DOT_KW)
        acc = jnp.maximum(acc, 0.0)
        o_ref[0] = acc[:, :128]
        o_ref[1] = acc[:, 128:]

    return pl.pallas_call(
        body,
        grid=(_NBLK,),
        in_specs=[
            pl.BlockSpec((2, _BN, 128), lambda i: (0, i, 0)),
            pl.BlockSpec((_BN, 128), lambda i: (i, 0)),
            pl.BlockSpec((128, 256), lambda i: (0, 0)),
            pl.BlockSpec((128, 256), lambda i: (0, 0)),
        ],
        out_specs=pl.BlockSpec((2, _BN, 128), lambda i: (0, i, 0)),
        out_shape=jax.ShapeDtypeStruct((2, _N, 128), jnp.float32),
    )(aggp, h0, wd, w)


def _tc_layer(aggs, hs, wd, w):
    def body(a_ref, h_ref, wd_ref, w_ref, o_ref):
        acc = _halves_matmul(a_ref, wd_ref)
        acc += _halves_matmul(h_ref, w_ref)
        acc = jnp.maximum(acc, 0.0)
        o_ref[0] = acc[:, :128]
        o_ref[1] = acc[:, 128:]

    return pl.pallas_call(
        body,
        grid=(_NBLK,),
        in_specs=[
            pl.BlockSpec((2, _BN, 128), lambda i: (0, i, 0)),
            pl.BlockSpec((2, _BN, 128), lambda i: (0, i, 0)),
            pl.BlockSpec((256, 256), lambda i: (0, 0)),
            pl.BlockSpec((256, 256), lambda i: (0, 0)),
        ],
        out_specs=pl.BlockSpec((2, _BN, 128), lambda i: (0, i, 0)),
        out_shape=jax.ShapeDtypeStruct((2, _N, 128), jnp.float32),
    )(aggs, hs, wd, w)


def _tc_layer_pool_head(aggs, hs, wd, w, batch2d, w1, b1, w2, b2):
    def body(a_ref, h_ref, wd_ref, w_ref, b_ref,
             w1_ref, b1_ref, w2_ref, b2_ref, o_ref, pool_scr):
        i = pl.program_id(0)

        acc = _halves_matmul(a_ref, wd_ref)
        acc += _halves_matmul(h_ref, w_ref)
        acc = jnp.maximum(acc, 0.0)

        @pl.when(i == 0)
        def _():
            pool_scr[...] = jnp.zeros_like(pool_scr)

        oh = (b_ref[...] == lax.broadcasted_iota(jnp.int32, (_BN, 64), 1))
        pool_scr[...] += lax.dot_general(oh.astype(jnp.float32), acc,
                                         (((0,), (0,)), ((), ())), **_DOT_KW)

        @pl.when(i == _NBLK - 1)
        def _():
            t = lax.dot_general(pool_scr[...], w1_ref[...], _DN,
                                **_DOT_KW) + b1_ref[...]
            t = jnp.maximum(t, 0.0)
            o_ref[...] = lax.dot_general(t, w2_ref[...], _DN,
                                         **_DOT_KW) + b2_ref[...]

    return pl.pallas_call(
        body,
        grid=(_NBLK,),
        in_specs=[
            pl.BlockSpec((2, _BN, 128), lambda i: (0, i, 0)),
            pl.BlockSpec((2, _BN, 128), lambda i: (0, i, 0)),
            pl.BlockSpec((256, 256), lambda i: (0, 0)),
            pl.BlockSpec((256, 256), lambda i: (0, 0)),
            pl.BlockSpec((_BN, 1), lambda i: (i, 0)),
            pl.BlockSpec((256, 256), lambda i: (0, 0)),
            pl.BlockSpec((1, 256), lambda i: (0, 0)),
            pl.BlockSpec((256, 10), lambda i: (0, 0)),
            pl.BlockSpec((1, 10), lambda i: (0, 0)),
        ],
        out_specs=pl.BlockSpec((64, 10), lambda i: (0, 0)),
        out_shape=jax.ShapeDtypeStruct((64, 10), jnp.float32),
        scratch_shapes=[pltpu.VMEM((64, 256), jnp.float32)],
    )(aggs, hs, wd, w, batch2d, w1, b1, w2, b2)


def kernel(x, lower_index, batch, W_down_0, W_0, W_down_1, W_1,
           W_down_2, W_2, lin1_w, lin1_b, lin2_w, lin2_b):
    src = lower_index[0]
    dst = lower_index[1]
    src2 = jnp.concatenate([src, src + _N]).reshape(2 * _CHUNKS, _CH)
    dst2 = dst.reshape(_CHUNKS, _CH)
    z128 = jnp.zeros((_N, 128), jnp.float32)
    batch2d = batch.reshape(_N, 1)
    b1 = lin1_b.reshape(1, 256)
    b2 = lin2_b.reshape(1, 10)

    seg_e = _sc_segment_sum(_N, edge_split=True)
    seg_f = _sc_segment_sum(2 * _N, edge_split=False)

    h0 = _tc_abs(x)
    a0 = seg_e(h0, src2, dst2, z128).reshape(2, _N, 128)
    hs1 = _tc_layer0(a0, h0, W_down_0, W_0)
    a1 = seg_f(hs1.reshape(2 * _N, 128), src2, dst2, z128).reshape(2, _N, 128)
    hs2 = _tc_layer(a1, hs1, W_down_1, W_1)
    a2 = seg_f(hs2.reshape(2 * _N, 128), src2, dst2, z128).reshape(2, _N, 128)
    return _tc_layer_pool_head(a2, hs2, W_down_2, W_2, batch2d,
                               lin1_w, b1, lin2_w, b2)


# speedup vs baseline: 1.1041x; 1.0035x over previous
import functools

import jax
import jax.numpy as jnp
from jax import lax
from jax.experimental import pallas as pl
from jax.experimental.pallas import tpu as pltpu
from jax.experimental.pallas import tpu_sc as plsc

_N = 10000
_E = 320000
_NS = 16
_CH = 125
_RPT = 624
_CHUNKS = _E // _CH
_BN = 2000
_NBLK = _N // _BN

_DOT_KW = dict(preferred_element_type=jnp.float32,
               precision=lax.Precision.DEFAULT)
_DN = (((1,), (0,)), ((), ()))


def _sc_segment_sum(hs_rows, edge_split):
    cpc = _CHUNKS // 2 if edge_split else _CHUNKS
    cpt = cpc // _NS
    ob = 8
    nb = cpt // ob
    mesh = plsc.VectorSubcoreMesh(core_axis_name="c", subcore_axis_name="s")

    @functools.partial(
        pl.kernel,
        out_type=jax.ShapeDtypeStruct((2 * _N, 128), jnp.float32),
        mesh=mesh,
        scratch_types=[
            pltpu.VMEM((2, ob, _CH), jnp.int32),
            pltpu.VMEM((2, ob, _CH), jnp.int32),
            pltpu.VMEM((_CH, 128), jnp.float32),
            pltpu.VMEM((_CH, 128), jnp.float32),
            pltpu.VMEM_SHARED((_N, 128), jnp.float32),
            pltpu.SemaphoreType.DMA,
            pltpu.SemaphoreType.DMA,
            pltpu.SemaphoreType.DMA,
            pltpu.SemaphoreType.DMA,
        ],
    )
    def seg(hs_hbm, src_hbm, dst_hbm, z_hbm, out_hbm,
            sidx2, didx2, rows0, rows1, acc, gsem0, gsem1, isem0, isem1):
        c = lax.axis_index("c")
        s = lax.axis_index("s")
        r0 = s * _RPT
        sbase = c * cpc + s * cpt
        dbase = (c * cpc + s * cpt) if edge_split else (s * cpt)
        rows = (rows0, rows1)
        gsems = (gsem0, gsem1)
        isems = (isem0, isem1)

        def stage(blk, p):
            pltpu.async_copy(src_hbm.at[pl.ds(sbase + blk * ob, ob)],
                             sidx2.at[p], isems[p])
            pltpu.async_copy(dst_hbm.at[pl.ds(dbase + blk * ob, ob)],
                             didx2.at[p], isems[p])

        def stage_wait(blk, p):
            pltpu.make_async_copy(src_hbm.at[pl.ds(sbase + blk * ob, ob)],
                                  sidx2.at[p], isems[p]).wait()
            pltpu.make_async_copy(dst_hbm.at[pl.ds(dbase + blk * ob, ob)],
                                  didx2.at[p], isems[p]).wait()

        def gather(p, j, jb):
            pltpu.async_copy(hs_hbm.at[sidx2.at[p].at[j]], rows[jb],
                             gsems[jb])

        def gather_wait(p, j, jb):
            pltpu.make_async_copy(hs_hbm.at[sidx2.at[p].at[j]], rows[jb],
                                  gsems[jb]).wait()

        stage(0, 0)
        stage_wait(0, 0)
        gather(0, 0, 0)
        gather(0, 1, 1)
        pltpu.sync_copy(z_hbm.at[pl.ds(r0, _RPT)], acc.at[pl.ds(r0, _RPT)])

        @pl.when(s == _NS - 1)
        def _():
            pltpu.sync_copy(z_hbm.at[pl.ds(_NS * _RPT, _N - _NS * _RPT)],
                            acc.at[pl.ds(_NS * _RPT, _N - _NS * _RPT)])

        plsc.subcore_barrier()

        def one_block(blk, p):
            dx = didx2.at[p]
            for j in range(ob):
                jb = j % 2
                if j == 0:
                    @pl.when(blk + 1 < nb)
                    def _():
                        stage(blk + 1, 1 - p)
                gather_wait(p, j, jb)
                pltpu.sync_copy(rows[jb], acc.at[dx.at[j]], add=True)
                if j + 2 < ob:
                    gather(p, j + 2, jb)
                else:
                    @pl.when(blk + 1 < nb)
                    def _():
                        if j + 2 == ob:
                            stage_wait(blk + 1, 1 - p)
                        gather(1 - p, j + 2 - ob, jb)

        @pl.loop(0, nb // 2)
        def _(m):
            one_block(2 * m, 0)
            one_block(2 * m + 1, 1)

        plsc.subcore_barrier()
        pltpu.sync_copy(acc.at[pl.ds(r0, _RPT)],
                        out_hbm.at[pl.ds(c * _N + r0, _RPT)])

        @pl.when(s == _NS - 1)
        def _():
            pltpu.sync_copy(
                acc.at[pl.ds(_NS * _RPT, _N - _NS * _RPT)],
                out_hbm.at[pl.ds(c * _N + _NS * _RPT, _N - _NS * _RPT)])

    def call(hs, src2, dst2, z):
        assert hs.shape == (hs_rows, 128)
        return seg(hs, src2, dst2, z)

    return call


def _tc_abs(x):
    def body(x_ref, o_ref):
        o_ref[...] = jnp.abs(x_ref[...])

    return pl.pallas_call(
        body,
        grid=(_NBLK,),
        in_specs=[pl.BlockSpec((_BN, 128), lambda i: (i, 0))],
        out_specs=pl.BlockSpec((_BN, 128), lambda i: (i, 0)),
        out_shape=jax.ShapeDtypeStruct((_N, 128), jnp.float32),
    )(x)


def _halves_matmul(a3, wref):
    acc = lax.dot_general(a3[0], wref[pl.ds(0, 128), :], _DN, **_DOT_KW)
    return acc + lax.dot_general(a3[1], wref[pl.ds(128, 128), :],
                                 _DN, **_DOT_KW)


def _tc_layer0(aggp, h0, wd, w):
    def body(a_ref, h_ref, wd_ref, w_ref, o_ref):
        agg = a_ref[0] + a_ref[1]
        acc = lax.dot_general(agg, wd_ref[...], _DN, **_DOT_KW)
        acc += lax.dot_general(h_ref[...], w_ref[...], _DN, **_DOT_KW)
        acc = jnp.maximum(acc, 0.0)
        o_ref[0] = acc[:, :128]
        o_ref[1] = acc[:, 128:]

    return pl.pallas_call(
        body,
        grid=(_NBLK,),
        in_specs=[
            pl.BlockSpec((2, _BN, 128), lambda i: (0, i, 0)),
            pl.BlockSpec((_BN, 128), lambda i: (i, 0)),
            pl.BlockSpec((128, 256), lambda i: (0, 0)),
            pl.BlockSpec((128, 256), lambda i: (0, 0)),
        ],
        out_specs=pl.BlockSpec((2, _BN, 128), lambda i: (0, i, 0)),
        out_shape=jax.ShapeDtypeStruct((2, _N, 128), jnp.float32),
    )(aggp, h0, wd, w)


def _tc_layer(aggs, hs, wd, w):
    def body(a_ref, h_ref, wd_ref, w_ref, o_ref):
        acc = _halves_matmul(a_ref, wd_ref)
        acc += _halves_matmul(h_ref, w_ref)
        acc = jnp.maximum(acc, 0.0)
        o_ref[0] = acc[:, :128]
        o_ref[1] = acc[:, 128:]

    return pl.pallas_call(
        body,
        grid=(_NBLK,),
        in_specs=[
            pl.BlockSpec((2, _BN, 128), lambda i: (0, i, 0)),
            pl.BlockSpec((2, _BN, 128), lambda i: (0, i, 0)),
            pl.BlockSpec((256, 256), lambda i: (0, 0)),
            pl.BlockSpec((256, 256), lambda i: (0, 0)),
        ],
        out_specs=pl.BlockSpec((2, _BN, 128), lambda i: (0, i, 0)),
        out_shape=jax.ShapeDtypeStruct((2, _N, 128), jnp.float32),
    )(aggs, hs, wd, w)


def _tc_layer_pool_head(aggs, hs, wd, w, batch2d, w1, b1, w2, b2):
    def body(a_ref, h_ref, wd_ref, w_ref, b_ref,
             w1_ref, b1_ref, w2_ref, b2_ref, o_ref, pool_scr):
        i = pl.program_id(0)

        acc = _halves_matmul(a_ref, wd_ref)
        acc += _halves_matmul(h_ref, w_ref)
        acc = jnp.maximum(acc, 0.0)

        @pl.when(i == 0)
        def _():
            pool_scr[...] = jnp.zeros_like(pool_scr)

        oh = (b_ref[...] == lax.broadcasted_iota(jnp.int32, (_BN, 64), 1))
        pool_scr[...] += lax.dot_general(oh.astype(jnp.float32), acc,
                                         (((0,), (0,)), ((), ())), **_DOT_KW)

        @pl.when(i == _NBLK - 1)
        def _():
            t = lax.dot_general(pool_scr[...], w1_ref[...], _DN,
                                **_DOT_KW) + b1_ref[...]
            t = jnp.maximum(t, 0.0)
            o_ref[...] = lax.dot_general(t, w2_ref[...], _DN,
                                         **_DOT_KW) + b2_ref[...]

    return pl.pallas_call(
        body,
        grid=(_NBLK,),
        in_specs=[
            pl.BlockSpec((2, _BN, 128), lambda i: (0, i, 0)),
            pl.BlockSpec((2, _BN, 128), lambda i: (0, i, 0)),
            pl.BlockSpec((256, 256), lambda i: (0, 0)),
            pl.BlockSpec((256, 256), lambda i: (0, 0)),
            pl.BlockSpec((_BN, 1), lambda i: (i, 0)),
            pl.BlockSpec((256, 256), lambda i: (0, 0)),
            pl.BlockSpec((1, 256), lambda i: (0, 0)),
            pl.BlockSpec((256, 10), lambda i: (0, 0)),
            pl.BlockSpec((1, 10), lambda i: (0, 0)),
        ],
        out_specs=pl.BlockSpec((64, 10), lambda i: (0, 0)),
        out_shape=jax.ShapeDtypeStruct((64, 10), jnp.float32),
        scratch_shapes=[pltpu.VMEM((64, 256), jnp.float32)],
    )(aggs, hs, wd, w, batch2d, w1, b1, w2, b2)


def kernel(x, lower_index, batch, W_down_0, W_0, W_down_1, W_1,
           W_down_2, W_2, lin1_w, lin1_b, lin2_w, lin2_b):
    src = lower_index[0]
    dst = lower_index[1]
    src2 = jnp.concatenate([src, src + _N]).reshape(2 * _CHUNKS, _CH)
    dst2 = dst.reshape(_CHUNKS, _CH)
    z128 = jnp.zeros((_N, 128), jnp.float32)
    batch2d = batch.reshape(_N, 1)
    b1 = lin1_b.reshape(1, 256)
    b2 = lin2_b.reshape(1, 10)

    seg_e = _sc_segment_sum(_N, edge_split=True)
    seg_f = _sc_segment_sum(2 * _N, edge_split=False)

    h0 = _tc_abs(x)
    a0 = seg_e(h0, src2, dst2, z128).reshape(2, _N, 128)
    hs1 = _tc_layer0(a0, h0, W_down_0, W_0)
    a1 = seg_f(hs1.reshape(2 * _N, 128), src2, dst2, z128).reshape(2, _N, 128)
    hs2 = _tc_layer(a1, hs1, W_down_1, W_1)
    a2 = seg_f(hs2.reshape(2 * _N, 128), src2, dst2, z128).reshape(2, _N, 128)
    return _tc_layer_pool_head(a2, hs2, W_down_2, W_2, batch2d,
                               lin1_w, b1, lin2_w, b2)
